# baseline (speedup 1.0000x reference)
import functools

import jax
import jax.numpy as jnp
from jax.experimental import pallas as pl
from jax.experimental.pallas import tpu as pltpu


def _conv3x3_kernel(x_ref, w_ref, b_ref, o_ref, *, C, H, W):
    HW = H * W
    xs = x_ref[0].astype(jnp.bfloat16).reshape(C, HW)
    P = W + 1
    padded = jnp.pad(xs, ((0, 0), (P, P)))
    col = jax.lax.broadcasted_iota(jnp.int32, (C, HW), 1) % W

    taps = []
    for kh in (0, 1, 2):
        for kw in (0, 1, 2):
            s = (kh - 1) * W + (kw - 1)
            t = padded[:, P + s: P + s + HW]
            if kw == 0:
                t = jnp.where(col != 0, t, 0)
            elif kw == 2:
                t = jnp.where(col != W - 1, t, 0)
            taps.append(t)
    patches = jnp.concatenate(taps, axis=0)

    acc = jnp.dot(w_ref[...], patches,
                  preferred_element_type=jnp.float32)
    acc = acc + b_ref[...]
    Cout = o_ref.shape[1]
    o_ref[0] = jnp.maximum(acc, 0.0).astype(o_ref.dtype).reshape(Cout, H, W)


def kernel(x, weight, bias):
    N, C, H, W = x.shape
    Cout = weight.shape[0]
    HW = H * W
    K = 9 * C

    wf = jnp.transpose(weight, (0, 2, 3, 1)).reshape(Cout, K).astype(jnp.bfloat16)
    b2 = bias.astype(jnp.float32).reshape(Cout, 1)

    out = pl.pallas_call(
        functools.partial(_conv3x3_kernel, C=C, H=H, W=W),
        out_shape=jax.ShapeDtypeStruct((N, Cout, H, W), x.dtype),
        grid=(N,),
        in_specs=[
            pl.BlockSpec((1, C, H, W), lambda n: (n, 0, 0, 0)),
            pl.BlockSpec((Cout, K), lambda n: (0, 0)),
            pl.BlockSpec((Cout, 1), lambda n: (0, 0)),
        ],
        out_specs=pl.BlockSpec((1, Cout, H, W), lambda n: (n, 0, 0, 0)),
        compiler_params=pltpu.CompilerParams(
            dimension_semantics=("parallel",),
            vmem_limit_bytes=64 * 1024 * 1024,
        ),
    )(x, wf, b2)
    return out


# speedup vs baseline: 1.7906x; 1.7906x over previous
import functools

import jax
import jax.numpy as jnp
from jax.experimental import pallas as pl
from jax.experimental.pallas import tpu as pltpu


def _conv3x3_kernel(x_ref, w_ref, b_ref, o_ref, *, C, H, W):
    HW = H * W
    xs = x_ref[0].astype(jnp.bfloat16).reshape(C, HW)
    P = W + 1
    padded = jnp.pad(xs, ((0, 0), (P, P)))
    col = jax.lax.broadcasted_iota(jnp.int32, (C, HW), 1) % W

    taps = []
    for kh in (0, 1, 2):
        for kw in (0, 1, 2):
            s = (kh - 1) * W + (kw - 1)
            t = padded[:, P + s: P + s + HW]
            if kw == 0:
                t = jnp.where(col != 0, t, 0)
            elif kw == 2:
                t = jnp.where(col != W - 1, t, 0)
            taps.append(t)
    patches = jnp.concatenate(taps, axis=0)

    acc = jax.lax.dot_general(
        patches, w_ref[...],
        dimension_numbers=(((0,), (1,)), ((), ())),
        preferred_element_type=jnp.float32)
    acc = acc + b_ref[...]
    Cout = o_ref.shape[-1]
    o_ref[0] = jnp.maximum(acc, 0.0).astype(o_ref.dtype).reshape(H, W, Cout)


def kernel(x, weight, bias):
    N, C, H, W = x.shape
    Cout = weight.shape[0]
    K = 9 * C

    wf = jnp.transpose(weight, (0, 2, 3, 1)).reshape(Cout, K).astype(jnp.bfloat16)
    b2 = bias.astype(jnp.float32).reshape(1, Cout)

    out = pl.pallas_call(
        functools.partial(_conv3x3_kernel, C=C, H=H, W=W),
        out_shape=jax.ShapeDtypeStruct((N, H, W, Cout), x.dtype),
        grid=(N,),
        in_specs=[
            pl.BlockSpec((1, C, H, W), lambda n: (n, 0, 0, 0)),
            pl.BlockSpec((Cout, K), lambda n: (0, 0)),
            pl.BlockSpec((1, Cout), lambda n: (0, 0)),
        ],
        out_specs=pl.BlockSpec((1, H, W, Cout), lambda n: (n, 0, 0, 0)),
        compiler_params=pltpu.CompilerParams(
            dimension_semantics=("parallel",),
            vmem_limit_bytes=64 * 1024 * 1024,
        ),
    )(x, wf, b2)
    return jnp.transpose(out, (0, 3, 1, 2))


# speedup vs baseline: 1.9957x; 1.1145x over previous
import functools

import jax
import jax.numpy as jnp
from jax.experimental import pallas as pl
from jax.experimental.pallas import tpu as pltpu


def _im2col(xs, *, C, H, W):
    HW = H * W
    P = W + 1
    padded = jnp.pad(xs, ((0, 0), (P, P)))
    col = jax.lax.broadcasted_iota(jnp.int32, (C, HW), 1) % W

    taps = []
    for kh in (0, 1, 2):
        for kw in (0, 1, 2):
            s = (kh - 1) * W + (kw - 1)
            t = padded[:, P + s: P + s + HW]
            if kw == 0:
                t = jnp.where(col != 0, t, 0)
            elif kw == 2:
                t = jnp.where(col != W - 1, t, 0)
            taps.append(t)
    return jnp.concatenate(taps, axis=0)


def _conv3x3_kernel(x_ref, w_ref, b_ref, o_ref, *, C, H, W, NB):
    HW = H * W
    patches = jnp.concatenate(
        [_im2col(x_ref[i].astype(jnp.bfloat16).reshape(C, HW), C=C, H=H, W=W)
         for i in range(NB)], axis=1)

    acc = jax.lax.dot_general(
        patches, w_ref[...],
        dimension_numbers=(((0,), (1,)), ((), ())),
        preferred_element_type=jnp.float32)
    acc = acc + b_ref[...]
    Cout = o_ref.shape[-1]
    res = jnp.maximum(acc, 0.0).astype(o_ref.dtype)
    for i in range(NB):
        o_ref[i] = res[i * HW:(i + 1) * HW].reshape(H, W, Cout)


def kernel(x, weight, bias):
    N, C, H, W = x.shape
    Cout = weight.shape[0]
    K = 9 * C

    wf = jnp.transpose(weight, (0, 2, 3, 1)).reshape(Cout, K).astype(jnp.bfloat16)
    b2 = bias.astype(jnp.float32).reshape(1, Cout)

    NB = 2 if N % 2 == 0 else 1
    out = pl.pallas_call(
        functools.partial(_conv3x3_kernel, C=C, H=H, W=W, NB=NB),
        out_shape=jax.ShapeDtypeStruct((N, H, W, Cout), x.dtype),
        grid=(N // NB,),
        in_specs=[
            pl.BlockSpec((NB, C, H, W), lambda n: (n, 0, 0, 0)),
            pl.BlockSpec((Cout, K), lambda n: (0, 0)),
            pl.BlockSpec((1, Cout), lambda n: (0, 0)),
        ],
        out_specs=pl.BlockSpec((NB, H, W, Cout), lambda n: (n, 0, 0, 0)),
        compiler_params=pltpu.CompilerParams(
            dimension_semantics=("parallel",),
            vmem_limit_bytes=64 * 1024 * 1024,
        ),
    )(x, wf, b2)
    return jnp.transpose(out, (0, 3, 1, 2))
